# baseline (speedup 1.0000x reference)
import functools
import math

import jax
import jax.numpy as jnp
from jax import lax
from jax.experimental import pallas as pl
from jax.experimental.pallas import tpu as pltpu
from jax.experimental.pallas import tpu_sc as plsc

_SC_NC = 2
_SC_NS = 16
_SC_NW = _SC_NC * _SC_NS

_N_POINTS = 20000
_MAX_K = 64
_SA1_RATIO = 0.1
_SA1_R = 0.2
_SA2_RATIO = 0.05
_SA2_R = 0.4


def _fps_body(px_ref, py_ref, pz_ref, out_ref, dists_ref, *, n_samples, n_valid):
    R, C = px_ref.shape
    px = px_ref[...]
    py = py_ref[...]
    pz = pz_ref[...]
    flat = (lax.broadcasted_iota(jnp.int32, (R, C), 0) * C
            + lax.broadcasted_iota(jnp.int32, (R, C), 1))
    validm = flat < n_valid
    x0 = px_ref[0, 0]
    y0 = py_ref[0, 0]
    z0 = pz_ref[0, 0]
    d0 = (px - x0) ** 2 + (py - y0) ** 2 + (pz - z0) ** 2
    dists_ref[...] = jnp.where(validm, d0, -jnp.inf)

    SR, SC_ = out_ref.shape
    slot = (lax.broadcasted_iota(jnp.int32, (SR, SC_), 0) * SC_
            + lax.broadcasted_iota(jnp.int32, (SR, SC_), 1))
    idxbuf0 = jnp.zeros((SR, SC_), jnp.int32)

    def body(i, idxbuf):
        dists = dists_ref[...]
        m = jnp.max(dists)
        nxt = jnp.min(jnp.where(dists == m, flat, jnp.int32(2 ** 30)))
        sel = flat == nxt
        cx = jnp.sum(jnp.where(sel, px, 0.0))
        cy = jnp.sum(jnp.where(sel, py, 0.0))
        cz = jnp.sum(jnp.where(sel, pz, 0.0))
        d = (px - cx) ** 2 + (py - cy) ** 2 + (pz - cz) ** 2
        dists_ref[...] = jnp.where(validm, jnp.minimum(dists, d), -jnp.inf)
        return jnp.where(slot == i, nxt, idxbuf)

    idxbuf = lax.fori_loop(1, n_samples, body, idxbuf0, unroll=False)
    out_ref[...] = idxbuf


def _fps(pos, n_samples):
    n = pos.shape[0]
    rows = -(-n // 128)
    npad = rows * 128
    pcols = jnp.pad(pos, ((0, npad - n), (0, 0)))
    px = pcols[:, 0].reshape(rows, 128)
    py = pcols[:, 1].reshape(rows, 128)
    pz = pcols[:, 2].reshape(rows, 128)
    srows = -(-n_samples // 128)
    out = pl.pallas_call(
        functools.partial(_fps_body, n_samples=n_samples, n_valid=n),
        out_shape=jax.ShapeDtypeStruct((srows, 128), jnp.int32),
        scratch_shapes=[pltpu.VMEM((rows, 128), jnp.float32)],
    )(px, py, pz)
    return out.reshape(-1)[:n_samples]


def _prep_layers(layers, pad_in):
    eps = 1e-05
    out = []
    for li, layer in enumerate(layers):
        w = layer['W']
        if li == 0 and w.shape[0] < pad_in:
            w = jnp.pad(w, ((0, pad_in - w.shape[0]), (0, 0)))
        s = (layer['gamma'] / jnp.sqrt(1.0 + eps))[None, :]
        out.append((w, layer['b'][None, :], s, layer['beta'][None, :]))
    return out


def _mlp_agg_body(a_ref, valid_ref,
                  w1, b1, s1, t1, w2, b2, s2, t2, w3, b3, s3, t3,
                  out_ref):
    layers = ((w1, b1, s1, t1), (w2, b2, s2, t2), (w3, b3, s3, t3))
    out = None
    for k in range(_MAX_K):
        h = a_ref[k]
        for (w, b, sc, tb) in layers:
            h = jnp.dot(h, w[...], preferred_element_type=jnp.float32) + b[...]
            h = jnp.maximum(h, 0.0) * sc[...] + tb[...]
        v = valid_ref[k] > 0.0
        h = jnp.where(v, h, -jnp.inf)
        out = h if out is None else jnp.maximum(out, h)
    out_ref[...] = jnp.where(out == -jnp.inf, 0.0, out)


def _mlp_agg(layers, msg, valid, nc_blk):
    nc = msg.shape[0]
    f = msg.shape[2]
    fp = -(-f // 8) * 8
    prep = _prep_layers(layers, fp)
    ncp = -(-nc // nc_blk) * nc_blk
    msg = jnp.pad(msg, ((0, ncp - nc), (0, 0), (0, fp - f)))
    validf = jnp.pad(valid.astype(jnp.float32), ((0, ncp - nc), (0, 0)))
    a = msg.transpose(1, 0, 2)
    validt = validf.T[:, :, None]
    co = prep[-1][0].shape[1]
    grid = ncp // nc_blk
    wspecs = []
    wargs = []
    for (w, b, s, t) in prep:
        for arr in (w, b, s, t):
            wspecs.append(pl.BlockSpec(arr.shape, lambda i: (0, 0)))
            wargs.append(arr)
    out = pl.pallas_call(
        _mlp_agg_body,
        grid=(grid,),
        in_specs=[
            pl.BlockSpec((_MAX_K, nc_blk, fp), lambda i: (0, i, 0)),
            pl.BlockSpec((_MAX_K, nc_blk, 1), lambda i: (0, i, 0)),
        ] + wspecs,
        out_specs=pl.BlockSpec((nc_blk, co), lambda i: (i, 0)),
        out_shape=jax.ShapeDtypeStruct((ncp, co), jnp.float32),
    )(a, validt, *wargs)
    return out[:nc]


def _head_body(a_ref, w1, b1, s1, t1, w2, b2, s2, t2, w3, b3, s3, t3,
               out_ref, *, n_valid):
    h = a_ref[...]
    for (w, b, sc, tb) in ((w1, b1, s1, t1), (w2, b2, s2, t2),
                           (w3, b3, s3, t3)):
        h = jnp.dot(h, w[...], preferred_element_type=jnp.float32) + b[...]
        h = jnp.maximum(h, 0.0) * sc[...] + tb[...]
    rows = lax.broadcasted_iota(jnp.int32, h.shape, 0)
    h = jnp.where(rows < n_valid, h, -jnp.inf)
    out_ref[...] = jnp.max(h, axis=0, keepdims=True)


def _head_mlp(layers, x):
    n, f = x.shape
    fp = -(-f // 8) * 8
    np_ = -(-n // 8) * 8
    prep = _prep_layers(layers, fp)
    xp = jnp.pad(x, ((0, np_ - n), (0, fp - f)))
    co = prep[-1][0].shape[1]
    wargs = [arr for lay in prep for arr in lay]
    out = pl.pallas_call(
        functools.partial(_head_body, n_valid=n),
        out_shape=jax.ShapeDtypeStruct((1, co), jnp.float32),
    )(xp, *wargs)
    return out


def _radius(pos_all, centers, r, max_k):
    d2 = (jnp.sum(centers ** 2, axis=1)[:, None]
          + jnp.sum(pos_all ** 2, axis=1)[None, :]
          - 2.0 * (centers @ pos_all.T))
    neg = jnp.where(d2 <= r * r, -d2, -jnp.inf)
    vals, idx = jax.lax.top_k(neg, max_k)
    valid = vals > -jnp.inf
    return idx, valid


def _sa_module(layers, x, pos, ratio, r, nc_blk):
    n = int(math.ceil(ratio * pos.shape[0]))
    idx = _fps(pos, n)
    centers = pos[idx]
    nbr, valid = _radius(pos, centers, r, _MAX_K)
    x_j = x[nbr]
    rel = pos[nbr] - centers[:, None, :]
    msg = jnp.concatenate([x_j, rel], axis=-1)
    out = _mlp_agg(layers, msg, valid, nc_blk)
    return out, centers


def kernel(p, params):
    x0 = p
    b0 = jnp.zeros((p.shape[0],), jnp.int32)
    x1, p1 = _sa_module(params['sa1'], x0, p, _SA1_RATIO, _SA1_R, 32)
    x2, p2 = _sa_module(params['sa2'], x1, p1, _SA2_RATIO, _SA2_R, 128)
    x3 = _head_mlp(params['sa3'], jnp.concatenate([x2, p2], axis=1))
    p3 = jnp.zeros((1, 3), jnp.float32)
    b1 = jnp.zeros((p1.shape[0],), jnp.int32)
    b2 = jnp.zeros((p2.shape[0],), jnp.int32)
    b3 = jnp.arange(1, dtype=jnp.int32)
    return (p, p1, p2, p3, x0, x1, x2, x3, b0, b1, b2, b3)


# speedup vs baseline: 1.4585x; 1.4585x over previous
import functools
import math

import jax
import jax.numpy as jnp
from jax import lax
from jax.experimental import pallas as pl
from jax.experimental.pallas import tpu as pltpu
from jax.experimental.pallas import tpu_sc as plsc

_SC_NC = 2
_SC_NS = 16
_SC_NW = _SC_NC * _SC_NS

_N_POINTS = 20000
_MAX_K = 64
_SA1_RATIO = 0.1
_SA1_R = 0.2
_SA2_RATIO = 0.05
_SA2_R = 0.4


def _fps_body(px_ref, py_ref, pz_ref, out_ref, dists_ref, *, n_samples, n_valid):
    R, C = px_ref.shape
    px = px_ref[...]
    py = py_ref[...]
    pz = pz_ref[...]
    flat = (lax.broadcasted_iota(jnp.int32, (R, C), 0) * C
            + lax.broadcasted_iota(jnp.int32, (R, C), 1))
    validm = flat < n_valid
    x0 = px_ref[0, 0]
    y0 = py_ref[0, 0]
    z0 = pz_ref[0, 0]
    d0 = (px - x0) ** 2 + (py - y0) ** 2 + (pz - z0) ** 2
    dists_ref[...] = jnp.where(validm, d0, -jnp.inf)

    SR, SC_ = out_ref.shape
    slot = (lax.broadcasted_iota(jnp.int32, (SR, SC_), 0) * SC_
            + lax.broadcasted_iota(jnp.int32, (SR, SC_), 1))
    idxbuf0 = jnp.zeros((SR, SC_), jnp.int32)

    def body(i, idxbuf):
        dists = dists_ref[...]
        m = jnp.max(dists)
        nxt = jnp.min(jnp.where(dists == m, flat, jnp.int32(2 ** 30)))
        sel = flat == nxt
        cx = jnp.sum(jnp.where(sel, px, 0.0))
        cy = jnp.sum(jnp.where(sel, py, 0.0))
        cz = jnp.sum(jnp.where(sel, pz, 0.0))
        d = (px - cx) ** 2 + (py - cy) ** 2 + (pz - cz) ** 2
        dists_ref[...] = jnp.where(validm, jnp.minimum(dists, d), -jnp.inf)
        return jnp.where(slot == i, nxt, idxbuf)

    idxbuf = lax.fori_loop(1, n_samples, body, idxbuf0, unroll=False)
    out_ref[...] = idxbuf


def _fps(pos, n_samples):
    n = pos.shape[0]
    rows = -(-n // 128)
    npad = rows * 128
    pcols = jnp.pad(pos, ((0, npad - n), (0, 0)))
    px = pcols[:, 0].reshape(rows, 128)
    py = pcols[:, 1].reshape(rows, 128)
    pz = pcols[:, 2].reshape(rows, 128)
    srows = -(-n_samples // 128)
    out = pl.pallas_call(
        functools.partial(_fps_body, n_samples=n_samples, n_valid=n),
        out_shape=jax.ShapeDtypeStruct((srows, 128), jnp.int32),
        scratch_shapes=[pltpu.VMEM((rows, 128), jnp.float32)],
    )(px, py, pz)
    return out.reshape(-1)[:n_samples]


def _prep_layers(layers, pad_in):
    eps = 1e-05
    out = []
    for li, layer in enumerate(layers):
        w = layer['W']
        if li == 0 and w.shape[0] < pad_in:
            w = jnp.pad(w, ((0, pad_in - w.shape[0]), (0, 0)))
        s = (layer['gamma'] / jnp.sqrt(1.0 + eps))[None, :]
        out.append((w, layer['b'][None, :], s, layer['beta'][None, :]))
    return out


def _mlp_agg_body(a_ref, valid_ref,
                  w1, b1, s1, t1, w2, b2, s2, t2, w3, b3, s3, t3,
                  out_ref):
    layers = ((w1, b1, s1, t1), (w2, b2, s2, t2), (w3, b3, s3, t3))
    out = None
    for k in range(_MAX_K):
        h = a_ref[k]
        for (w, b, sc, tb) in layers:
            h = jnp.dot(h, w[...], preferred_element_type=jnp.float32) + b[...]
            h = jnp.maximum(h, 0.0) * sc[...] + tb[...]
        v = valid_ref[k] > 0.0
        h = jnp.where(v, h, -jnp.inf)
        out = h if out is None else jnp.maximum(out, h)
    out_ref[...] = jnp.where(out == -jnp.inf, 0.0, out)


def _mlp_agg(layers, msg, valid, nc_blk):
    nc = msg.shape[1]
    f = msg.shape[2]
    fp = -(-f // 8) * 8
    prep = _prep_layers(layers, fp)
    ncp = -(-nc // nc_blk) * nc_blk
    a = jnp.pad(msg, ((0, 0), (0, ncp - nc), (0, fp - f)))
    validt = jnp.pad(valid.astype(jnp.float32), ((0, 0), (0, ncp - nc)))[:, :, None]
    co = prep[-1][0].shape[1]
    grid = ncp // nc_blk
    wspecs = []
    wargs = []
    for (w, b, s, t) in prep:
        for arr in (w, b, s, t):
            wspecs.append(pl.BlockSpec(arr.shape, lambda i: (0, 0)))
            wargs.append(arr)
    out = pl.pallas_call(
        _mlp_agg_body,
        grid=(grid,),
        in_specs=[
            pl.BlockSpec((_MAX_K, nc_blk, fp), lambda i: (0, i, 0)),
            pl.BlockSpec((_MAX_K, nc_blk, 1), lambda i: (0, i, 0)),
        ] + wspecs,
        out_specs=pl.BlockSpec((nc_blk, co), lambda i: (i, 0)),
        out_shape=jax.ShapeDtypeStruct((ncp, co), jnp.float32),
    )(a, validt, *wargs)
    return out[:nc]


def _head_body(a_ref, w1, b1, s1, t1, w2, b2, s2, t2, w3, b3, s3, t3,
               out_ref, *, n_valid):
    h = a_ref[...]
    for (w, b, sc, tb) in ((w1, b1, s1, t1), (w2, b2, s2, t2),
                           (w3, b3, s3, t3)):
        h = jnp.dot(h, w[...], preferred_element_type=jnp.float32) + b[...]
        h = jnp.maximum(h, 0.0) * sc[...] + tb[...]
    rows = lax.broadcasted_iota(jnp.int32, h.shape, 0)
    h = jnp.where(rows < n_valid, h, -jnp.inf)
    out_ref[...] = jnp.max(h, axis=0, keepdims=True)


def _head_mlp(layers, x):
    n, f = x.shape
    fp = -(-f // 8) * 8
    np_ = -(-n // 8) * 8
    prep = _prep_layers(layers, fp)
    xp = jnp.pad(x, ((0, np_ - n), (0, fp - f)))
    co = prep[-1][0].shape[1]
    wargs = [arr for lay in prep for arr in lay]
    out = pl.pallas_call(
        functools.partial(_head_body, n_valid=n),
        out_shape=jax.ShapeDtypeStruct((1, co), jnp.float32),
    )(xp, *wargs)
    return out


def _radius(pos_all, centers, r, max_k):
    d2 = (jnp.sum(centers ** 2, axis=1)[:, None]
          + jnp.sum(pos_all ** 2, axis=1)[None, :]
          - 2.0 * (centers @ pos_all.T))
    neg = jnp.where(d2 <= r * r, -d2, -jnp.inf)
    vals, idx = jax.lax.top_k(neg, max_k)
    valid = vals > -jnp.inf
    return idx, valid


def _sa_module(layers, x, pos, ratio, r, nc_blk):
    n = int(math.ceil(ratio * pos.shape[0]))
    idx = _fps(pos, n)
    centers = pos[idx]
    nbr, valid = _radius(pos, centers, r, _MAX_K)
    nbr_t = nbr.T
    x_jt = x[nbr_t]
    rel_t = pos[nbr_t] - centers[None, :, :]
    msg_t = jnp.concatenate([x_jt, rel_t], axis=-1)
    out = _mlp_agg(layers, msg_t, valid.T, nc_blk)
    return out, centers


def kernel(p, params):
    x0 = p
    b0 = jnp.zeros((p.shape[0],), jnp.int32)
    x1, p1 = _sa_module(params['sa1'], x0, p, _SA1_RATIO, _SA1_R, 32)
    x2, p2 = _sa_module(params['sa2'], x1, p1, _SA2_RATIO, _SA2_R, 128)
    x3 = _head_mlp(params['sa3'], jnp.concatenate([x2, p2], axis=1))
    p3 = jnp.zeros((1, 3), jnp.float32)
    b1 = jnp.zeros((p1.shape[0],), jnp.int32)
    b2 = jnp.zeros((p2.shape[0],), jnp.int32)
    b3 = jnp.arange(1, dtype=jnp.int32)
    return (p, p1, p2, p3, x0, x1, x2, x3, b0, b1, b2, b3)
